# Initial kernel scaffold; baseline (speedup 1.0000x reference)
import jax
import jax.numpy as jnp
from jax.experimental import pallas as pl


def kernel(edge_index, x, Wh, bh, a, ba, Wo, bo):
    raise NotImplementedError("write your pallas kernel here")


# speedup vs baseline: 13.1924x; 13.1924x over previous
import functools

import jax
import jax.numpy as jnp
from jax import lax
from jax.experimental import pallas as pl
from jax.experimental.pallas import tpu as pltpu
from jax.experimental.pallas import tpu_sc as plsc

N = 10000
DX = 128
DH = 64
K = 4
E = 320000

NTILE = 16
NCORE = 2
C = 128
EL = E + N
PT = 20736
CHUNKS = PT // C
EPAD = PT * NTILE
RW = 144
NP = 10112
ZR = NP // NTILE

_F32 = jnp.float32
_HI = jax.lax.Precision.HIGHEST


def _tc1(x_ref, w_ref, b_ref, am_ref, h2_ref, aux_ref, t_ref):
    h = jnp.dot(x_ref[...], w_ref[...], precision=_HI,
                preferred_element_type=_F32) + b_ref[...]
    h2_ref[0:N, :] = h[:, 0:128]
    h2_ref[N:2 * N, :] = h[:, 128:256]
    aux = jnp.dot(h, am_ref[...], precision=_HI, preferred_element_type=_F32)
    aux_ref[0:N, :] = aux
    aux_ref[N:NP, :] = jnp.zeros((NP - N, 8), _F32)
    t_ref[...] = jnp.max(aux, axis=0, keepdims=True)


def _sc_body(src_ref, dst_ref, ad_ref, as_ref, h2_ref, bap_ref, tp_ref,
             z_ref, out_ref, srcb, dstb, offb, auxd, auxs, hrow,
             stage, bap, tp, acc, sem):
    c = lax.axis_index("c")
    s = lax.axis_index("s")
    pltpu.sync_copy(z_ref, acc.at[pl.ds(s * ZR, ZR)])
    pltpu.sync_copy(bap_ref, bap)
    pltpu.sync_copy(tp_ref, tp)
    plsc.subcore_barrier()

    ebase = s * PT
    ii = lax.iota(jnp.int32, 16)

    def run(hoff, hbase):
        tailmask = (ii == hoff) | (ii == hoff + 1)

        def chunk_body(g, carry):
            base = ebase + g * C
            pltpu.sync_copy(src_ref.at[pl.ds(base, C)], srcb.at[0])
            pltpu.sync_copy(dst_ref.at[pl.ds(base, C)], dstb.at[0])
            for r in range(C // 16):
                offb[0, pl.ds(r * 16, 16)] = srcb[0, pl.ds(r * 16, 16)] + hbase
            pltpu.async_copy(ad_ref.at[dstb.at[0]], auxd, sem).wait()
            pltpu.async_copy(as_ref.at[srcb.at[0]], auxs, sem).wait()
            pltpu.async_copy(h2_ref.at[offb.at[0]], hrow, sem).wait()

            def edge_body(j, carry2):
                z = auxd[j, :] + auxs[j, :] + bap[:]
                z = jnp.where(z > 0, z, 0.01 * z) - tp[:]
                ex = jnp.exp(z)
                ea = ex[hoff]
                eb = ex[hoff + 1]
                for r in range(4):
                    stage[j, pl.ds(r * 16, 16)] = (
                        hrow[j, pl.ds(r * 16, 16)] * ea)
                for r in range(4, 8):
                    stage[j, pl.ds(r * 16, 16)] = (
                        hrow[j, pl.ds(r * 16, 16)] * eb)
                stage[j, pl.ds(128, 16)] = jnp.where(tailmask, ex, 0.0)
                return carry2

            lax.fori_loop(0, C, edge_body, 0)
            pltpu.sync_copy(stage, acc.at[dstb.at[0]], add=True)
            return carry

        lax.fori_loop(0, CHUNKS, chunk_body, 0)

    pl.when(c == 0)(lambda: run(0, 0))
    pl.when(c == 1)(lambda: run(2, N))
    plsc.subcore_barrier()
    pltpu.sync_copy(acc.at[pl.ds(s * ZR, ZR)],
                    out_ref.at[c, pl.ds(s * ZR, ZR)])


_sc_edges = functools.partial(
    pl.kernel,
    out_type=jax.ShapeDtypeStruct((NCORE, NP, RW), _F32),
    mesh=plsc.VectorSubcoreMesh(core_axis_name="c", subcore_axis_name="s"),
    scratch_types=[
        pltpu.VMEM((1, C), jnp.int32),
        pltpu.VMEM((1, C), jnp.int32),
        pltpu.VMEM((1, C), jnp.int32),
        pltpu.VMEM((C, 16), _F32),
        pltpu.VMEM((C, 16), _F32),
        pltpu.VMEM((C, 128), _F32),
        pltpu.VMEM((C, RW), _F32),
        pltpu.VMEM((16,), _F32),
        pltpu.VMEM((16,), _F32),
        pltpu.VMEM_SHARED((NP, RW), _F32),
        pltpu.SemaphoreType.DMA,
    ],
    compiler_params=pltpu.CompilerParams(use_tc_tiling_on_sc=False),
)(_sc_body)


def _tc2(sc_ref, wo_ref, bo_ref, o_ref):
    parts = []
    for k in range(K):
        c = k // 2
        p = k % 2
        numer = sc_ref[c, 0:N, p * 64:(p + 1) * 64]
        scol = 128 + 2 * c + p
        s = sc_ref[c, 0:N, scol:scol + 1]
        parts.append(numer / s)
    aggr = jnp.concatenate(parts, axis=-1)
    mx = jnp.max(aggr, axis=0, keepdims=True)
    ep = jnp.exp(aggr - mx)
    soft = ep / jnp.sum(ep, axis=0, keepdims=True)
    o_ref[...] = jnp.dot(soft, wo_ref[...], precision=_HI,
                         preferred_element_type=_F32) + bo_ref[...]


def kernel(edge_index, x, Wh, bh, a, ba, Wo, bo):
    loops = jnp.arange(N, dtype=edge_index.dtype)
    src = jnp.concatenate([edge_index[0], loops])
    dst = jnp.concatenate([edge_index[1], loops])
    src_p = jnp.concatenate([src, jnp.zeros((EPAD - EL,), jnp.int32)])
    dst_p = jnp.concatenate([dst, jnp.full((EPAD - EL,), N, jnp.int32)])

    W1 = Wh.transpose(2, 0, 1).reshape(DX, K * DH)
    b1 = bh.reshape(1, K * DH)
    am = jnp.zeros((K * DH, 8), _F32)
    for k in range(K):
        am = am.at[k * DH:(k + 1) * DH, k].set(a[k, :DH])
        am = am.at[k * DH:(k + 1) * DH, 4 + k].set(a[k, DH:])

    h2, aux, t8 = pl.pallas_call(
        _tc1,
        out_shape=[
            jax.ShapeDtypeStruct((2 * N, 128), _F32),
            jax.ShapeDtypeStruct((NP, 8), _F32),
            jax.ShapeDtypeStruct((1, 8), _F32),
        ],
    )(x, W1, b1, am)

    adst16 = jnp.tile(aux[:, 0:4], (1, 4))
    asrc16 = jnp.tile(aux[:, 4:8], (1, 4))
    tk = t8[0, :4] + t8[0, 4:8] + ba
    tk = jnp.where(tk > 0, tk, 0.01 * tk)
    tpat = jnp.tile(tk, 4)
    bapat = jnp.tile(ba, 4)
    zrows = jnp.zeros((ZR, RW), _F32)

    sc_out = _sc_edges(src_p, dst_p, adst16, asrc16, h2, bapat, tpat, zrows)

    return pl.pallas_call(
        _tc2,
        out_shape=jax.ShapeDtypeStruct((N, DH), _F32),
    )(sc_out, Wo.T, bo[None, :])


# speedup vs baseline: 17.1467x; 1.2997x over previous
import functools

import jax
import jax.numpy as jnp
from jax import lax
from jax.experimental import pallas as pl
from jax.experimental.pallas import tpu as pltpu
from jax.experimental.pallas import tpu_sc as plsc

N = 10000
DX = 128
DH = 64
K = 4
E = 320000

NTILE = 16
NCORE = 2
C = 64
EL = E + N
PT = 20736
CHUNKS = PT // C
EPAD = PT * NTILE
RW = 144
NP = 10112
ZR = NP // NTILE

_F32 = jnp.float32
_HI = jax.lax.Precision.HIGHEST


def _tc1(x_ref, w_ref, b_ref, am_ref, h2_ref, aux_ref, t_ref):
    h = jnp.dot(x_ref[...], w_ref[...], precision=_HI,
                preferred_element_type=_F32) + b_ref[...]
    h2_ref[0:N, :] = h[:, 0:128]
    h2_ref[N:2 * N, :] = h[:, 128:256]
    aux = jnp.dot(h, am_ref[...], precision=_HI, preferred_element_type=_F32)
    aux_ref[0:N, :] = aux
    aux_ref[N:NP, :] = jnp.zeros((NP - N, 8), _F32)
    t_ref[...] = jnp.max(aux, axis=0, keepdims=True)


def _sc_body(idx_ref, ad_ref, as_ref, h2_ref, bap_ref, tp_ref,
             out_ref, idxA, idxB, offA, offB, auxdA, auxsA, auxdB, auxsB,
             hrowA, hrowB, stage, bap, tp, acc, gsA, gsB, ssem):
    c = lax.axis_index("c")
    s = lax.axis_index("s")

    def zrow_body(j, carry):
        for r in range(RW // 16):
            stage[j, pl.ds(r * 16, 16)] = jnp.zeros((16,), _F32)
        return carry

    lax.fori_loop(0, C, zrow_body, 0)
    for i in range(ZR // C):
        pltpu.sync_copy(stage, acc.at[pl.ds(s * ZR + i * C, C)])
    rem = ZR % C
    if rem:
        pltpu.sync_copy(stage.at[pl.ds(0, rem)],
                        acc.at[pl.ds(s * ZR + (ZR // C) * C, rem)])
    pltpu.sync_copy(bap_ref, bap)
    pltpu.sync_copy(tp_ref, tp)
    plsc.subcore_barrier()

    ii = lax.iota(jnp.int32, 16)

    def run(hoff, hbase):
        srow, drow = 0, 1
        tailmask = (ii == hoff) | (ii == hoff + 1)

        def load_idx(g, buf, off):
            pltpu.sync_copy(idx_ref.at[pl.ds((s * CHUNKS + g) * 2, 2)], buf)
            for r in range(C // 16):
                off[0, pl.ds(r * 16, 16)] = buf[srow, pl.ds(r * 16, 16)] + hbase

        def issue_gathers(buf, off, auxd, auxs, hrow, sem):
            pltpu.async_copy(ad_ref.at[buf.at[drow]], auxd, sem)
            pltpu.async_copy(as_ref.at[buf.at[srow]], auxs, sem)
            pltpu.async_copy(h2_ref.at[off.at[0]], hrow, sem)

        def wait_gathers(buf, off, auxd, auxs, hrow, sem):
            pltpu.make_async_copy(ad_ref.at[buf.at[drow]], auxd, sem).wait()
            pltpu.make_async_copy(as_ref.at[buf.at[srow]], auxs, sem).wait()
            pltpu.make_async_copy(h2_ref.at[off.at[0]], hrow, sem).wait()

        def compute(buf, auxd, auxs, hrow):
            def edge_body(j, carry2):
                z = auxd[j, :] + auxs[j, :] + bap[:]
                z = jnp.where(z > 0, z, 0.01 * z) - tp[:]
                ex = jnp.exp(z)
                ea = ex[hoff]
                eb = ex[hoff + 1]
                for r in range(4):
                    stage[j, pl.ds(r * 16, 16)] = (
                        hrow[j, pl.ds(r * 16, 16)] * ea)
                for r in range(4, 8):
                    stage[j, pl.ds(r * 16, 16)] = (
                        hrow[j, pl.ds(r * 16, 16)] * eb)
                stage[j, pl.ds(128, 16)] = jnp.where(tailmask, ex, 0.0)
                return carry2

            lax.fori_loop(0, C, edge_body, 0)
            pltpu.async_copy(stage, acc.at[buf.at[drow]], ssem, add=True)

        def wait_scatter():
            pltpu.make_async_copy(stage, acc.at[idxA.at[drow]], ssem).wait()

        load_idx(0, idxA, offA)
        issue_gathers(idxA, offA, auxdA, auxsA, hrowA, gsA)

        def pair_body(p, carry):
            wait_gathers(idxA, offA, auxdA, auxsA, hrowA, gsA)
            pl.when(p > 0)(wait_scatter)
            load_idx(2 * p + 1, idxB, offB)
            issue_gathers(idxB, offB, auxdB, auxsB, hrowB, gsB)
            compute(idxA, auxdA, auxsA, hrowA)
            wait_gathers(idxB, offB, auxdB, auxsB, hrowB, gsB)
            wait_scatter()

            def prefetch_a():
                load_idx(2 * p + 2, idxA, offA)
                issue_gathers(idxA, offA, auxdA, auxsA, hrowA, gsA)

            pl.when(p < CHUNKS // 2 - 1)(prefetch_a)
            compute(idxB, auxdB, auxsB, hrowB)
            return carry

        lax.fori_loop(0, CHUNKS // 2, pair_body, 0)
        wait_scatter()

    pl.when(c == 0)(lambda: run(0, 0))
    pl.when(c == 1)(lambda: run(2, N))
    plsc.subcore_barrier()
    pltpu.sync_copy(acc.at[pl.ds(s * ZR, ZR)],
                    out_ref.at[c, pl.ds(s * ZR, ZR)])


_sc_edges = functools.partial(
    pl.kernel,
    out_type=jax.ShapeDtypeStruct((NCORE, NP, RW), _F32),
    mesh=plsc.VectorSubcoreMesh(core_axis_name="c", subcore_axis_name="s"),
    scratch_types=[
        pltpu.VMEM((2, C), jnp.int32),
        pltpu.VMEM((2, C), jnp.int32),
        pltpu.VMEM((1, C), jnp.int32),
        pltpu.VMEM((1, C), jnp.int32),
        pltpu.VMEM((C, 16), _F32),
        pltpu.VMEM((C, 16), _F32),
        pltpu.VMEM((C, 16), _F32),
        pltpu.VMEM((C, 16), _F32),
        pltpu.VMEM((C, 128), _F32),
        pltpu.VMEM((C, 128), _F32),
        pltpu.VMEM((C, RW), _F32),
        pltpu.VMEM((16,), _F32),
        pltpu.VMEM((16,), _F32),
        pltpu.VMEM_SHARED((NP, RW), _F32),
        pltpu.SemaphoreType.DMA,
        pltpu.SemaphoreType.DMA,
        pltpu.SemaphoreType.DMA,
    ],
    compiler_params=pltpu.CompilerParams(use_tc_tiling_on_sc=False),
)(_sc_body)


def _tc2(sc_ref, wo_ref, bo_ref, o_ref):
    parts = []
    for k in range(K):
        c = k // 2
        p = k % 2
        numer = sc_ref[c, 0:N, p * 64:(p + 1) * 64]
        scol = 128 + 2 * c + p
        s = sc_ref[c, 0:N, scol:scol + 1]
        parts.append(numer / s)
    aggr = jnp.concatenate(parts, axis=-1)
    mx = jnp.max(aggr, axis=0, keepdims=True)
    ep = jnp.exp(aggr - mx)
    soft = ep / jnp.sum(ep, axis=0, keepdims=True)
    o_ref[...] = jnp.dot(soft, wo_ref[...], precision=_HI,
                         preferred_element_type=_F32) + bo_ref[...]


def kernel(edge_index, x, Wh, bh, a, ba, Wo, bo):
    loops = jnp.arange(N, dtype=edge_index.dtype)
    src = jnp.concatenate([edge_index[0], loops])
    dst = jnp.concatenate([edge_index[1], loops])
    src_p = jnp.concatenate([src, jnp.zeros((EPAD - EL,), jnp.int32)])
    dst_p = jnp.concatenate([dst, jnp.full((EPAD - EL,), N, jnp.int32)])
    src_c = src_p.reshape(-1, C)
    dst_c = dst_p.reshape(-1, C)
    idx2 = jnp.stack([src_c, dst_c], axis=1).reshape(-1, C)

    W1 = Wh.transpose(2, 0, 1).reshape(DX, K * DH)
    b1 = bh.reshape(1, K * DH)
    am = jnp.zeros((K * DH, 8), _F32)
    for k in range(K):
        am = am.at[k * DH:(k + 1) * DH, k].set(a[k, :DH])
        am = am.at[k * DH:(k + 1) * DH, 4 + k].set(a[k, DH:])

    h2, aux, t8 = pl.pallas_call(
        _tc1,
        out_shape=[
            jax.ShapeDtypeStruct((2 * N, 128), _F32),
            jax.ShapeDtypeStruct((NP, 8), _F32),
            jax.ShapeDtypeStruct((1, 8), _F32),
        ],
    )(x, W1, b1, am)

    adst16 = jnp.tile(aux[:, 0:4], (1, 4))
    asrc16 = jnp.tile(aux[:, 4:8], (1, 4))
    tk = t8[0, :4] + t8[0, 4:8] + ba
    tk = jnp.where(tk > 0, tk, 0.01 * tk)
    tpat = jnp.tile(tk, 4)
    bapat = jnp.tile(ba, 4)

    sc_out = _sc_edges(idx2, adst16, asrc16, h2, bapat, tpat)

    return pl.pallas_call(
        _tc2,
        out_shape=jax.ShapeDtypeStruct((N, DH), _F32),
    )(sc_out, Wo.T, bo[None, :])


# speedup vs baseline: 18.5771x; 1.0834x over previous
import functools

import jax
import jax.numpy as jnp
from jax import lax
from jax.experimental import pallas as pl
from jax.experimental.pallas import tpu as pltpu
from jax.experimental.pallas import tpu_sc as plsc

N = 10000
DX = 128
DH = 64
K = 4
E = 320000

NTILE = 16
NCORE = 2
C = 64
EL = E + N
PT = 20736
CHUNKS = PT // C
EPAD = PT * NTILE
RW = 144
NP = 10112
ZR = NP // NTILE

_F32 = jnp.float32
_HI = jax.lax.Precision.HIGHEST


def _tc1(x_ref, w_ref, b_ref, am_ref, h2_ref, aux_ref, t_ref):
    h = jnp.dot(x_ref[...], w_ref[...], precision=_HI,
                preferred_element_type=_F32) + b_ref[...]
    h2_ref[0:N, :] = h[:, 0:128]
    h2_ref[N:2 * N, :] = h[:, 128:256]
    aux = jnp.dot(h, am_ref[...], precision=_HI, preferred_element_type=_F32)
    aux_ref[0:N, :] = aux
    aux_ref[N:NP, :] = jnp.zeros((NP - N, 8), _F32)
    t_ref[...] = jnp.max(aux, axis=0, keepdims=True)


def _sc_body(idx_ref, ad_ref, as_ref, h2_ref, bap_ref, tp_ref,
             out_ref, idxA, idxB, offA, offB, sidxA, sidxB,
             auxdA, auxsA, auxdB, auxsB,
             hrowA, hrowB, stageA, stageB, bap, tp, acc,
             gsA, gsB, ssA, ssB):
    c = lax.axis_index("c")
    s = lax.axis_index("s")

    def zrow_body(j, carry):
        for r in range(RW // 16):
            stageA[j, pl.ds(r * 16, 16)] = jnp.zeros((16,), _F32)
        return carry

    lax.fori_loop(0, C, zrow_body, 0)
    for i in range(ZR // C):
        pltpu.sync_copy(stageA, acc.at[pl.ds(s * ZR + i * C, C)])
    rem = ZR % C
    if rem:
        pltpu.sync_copy(stageA.at[pl.ds(0, rem)],
                        acc.at[pl.ds(s * ZR + (ZR // C) * C, rem)])
    pltpu.sync_copy(bap_ref, bap)
    pltpu.sync_copy(tp_ref, tp)
    plsc.subcore_barrier()

    ii = lax.iota(jnp.int32, 16)

    def run(hoff, hbase):
        srow, drow = 0, 1
        tailmask = (ii == hoff) | (ii == hoff + 1)

        def load_idx(g, buf, off):
            pltpu.sync_copy(idx_ref.at[pl.ds((s * CHUNKS + g) * 2, 2)], buf)
            for r in range(C // 16):
                off[0, pl.ds(r * 16, 16)] = buf[srow, pl.ds(r * 16, 16)] + hbase

        def issue_gathers(buf, off, auxd, auxs, hrow, sem):
            pltpu.async_copy(h2_ref.at[off.at[0]], hrow, sem)
            pltpu.async_copy(ad_ref.at[buf.at[drow]], auxd, sem)
            pltpu.async_copy(as_ref.at[buf.at[srow]], auxs, sem)

        def wait_gathers(buf, off, auxd, auxs, hrow, sem):
            pltpu.make_async_copy(ad_ref.at[buf.at[drow]], auxd, sem).wait()
            pltpu.make_async_copy(as_ref.at[buf.at[srow]], auxs, sem).wait()
            pltpu.make_async_copy(h2_ref.at[off.at[0]], hrow, sem).wait()

        def compute(p, buf, sidx, auxd, auxs, hrow, stage, ssem):
            pl.when(p > 0)(
                lambda: pltpu.make_async_copy(
                    stage, acc.at[sidx.at[0]], ssem).wait())
            for r in range(C // 16):
                sidx[0, pl.ds(r * 16, 16)] = buf[drow, pl.ds(r * 16, 16)]

            def edge_body(j, carry2):
                z = auxd[j, :] + auxs[j, :] + bap[:]
                z = jnp.where(z > 0, z, 0.01 * z) - tp[:]
                ex = jnp.exp(z)
                ea = ex[hoff]
                eb = ex[hoff + 1]
                for r in range(4):
                    stage[j, pl.ds(r * 16, 16)] = (
                        hrow[j, pl.ds(r * 16, 16)] * ea)
                for r in range(4, 8):
                    stage[j, pl.ds(r * 16, 16)] = (
                        hrow[j, pl.ds(r * 16, 16)] * eb)
                stage[j, pl.ds(128, 16)] = jnp.where(tailmask, ex, 0.0)
                return carry2

            lax.fori_loop(0, C, edge_body, 0, unroll=2)
            pltpu.async_copy(stage, acc.at[sidx.at[0]], ssem, add=True)

        load_idx(0, idxA, offA)
        issue_gathers(idxA, offA, auxdA, auxsA, hrowA, gsA)

        def pair_body(p, carry):
            wait_gathers(idxA, offA, auxdA, auxsA, hrowA, gsA)
            load_idx(2 * p + 1, idxB, offB)
            issue_gathers(idxB, offB, auxdB, auxsB, hrowB, gsB)
            compute(p, idxA, sidxA, auxdA, auxsA, hrowA, stageA, ssA)
            wait_gathers(idxB, offB, auxdB, auxsB, hrowB, gsB)

            def prefetch_a():
                load_idx(2 * p + 2, idxA, offA)
                issue_gathers(idxA, offA, auxdA, auxsA, hrowA, gsA)

            pl.when(p < CHUNKS // 2 - 1)(prefetch_a)
            compute(p, idxB, sidxB, auxdB, auxsB, hrowB, stageB, ssB)
            return carry

        lax.fori_loop(0, CHUNKS // 2, pair_body, 0)
        pltpu.make_async_copy(stageA, acc.at[sidxA.at[0]], ssA).wait()
        pltpu.make_async_copy(stageB, acc.at[sidxB.at[0]], ssB).wait()

    pl.when(c == 0)(lambda: run(0, 0))
    pl.when(c == 1)(lambda: run(2, N))
    plsc.subcore_barrier()
    pltpu.sync_copy(acc.at[pl.ds(s * ZR, ZR)],
                    out_ref.at[c, pl.ds(s * ZR, ZR)])


_sc_edges = functools.partial(
    pl.kernel,
    out_type=jax.ShapeDtypeStruct((NCORE, NP, RW), _F32),
    mesh=plsc.VectorSubcoreMesh(core_axis_name="c", subcore_axis_name="s"),
    scratch_types=[
        pltpu.VMEM((2, C), jnp.int32),
        pltpu.VMEM((2, C), jnp.int32),
        pltpu.VMEM((1, C), jnp.int32),
        pltpu.VMEM((1, C), jnp.int32),
        pltpu.VMEM((1, C), jnp.int32),
        pltpu.VMEM((1, C), jnp.int32),
        pltpu.VMEM((C, 16), _F32),
        pltpu.VMEM((C, 16), _F32),
        pltpu.VMEM((C, 16), _F32),
        pltpu.VMEM((C, 16), _F32),
        pltpu.VMEM((C, 128), _F32),
        pltpu.VMEM((C, 128), _F32),
        pltpu.VMEM((C, RW), _F32),
        pltpu.VMEM((C, RW), _F32),
        pltpu.VMEM((16,), _F32),
        pltpu.VMEM((16,), _F32),
        pltpu.VMEM_SHARED((NP, RW), _F32),
        pltpu.SemaphoreType.DMA,
        pltpu.SemaphoreType.DMA,
        pltpu.SemaphoreType.DMA,
        pltpu.SemaphoreType.DMA,
    ],
    compiler_params=pltpu.CompilerParams(use_tc_tiling_on_sc=False),
)(_sc_body)


def _tc2(sc_ref, wo_ref, bo_ref, o_ref):
    parts = []
    for k in range(K):
        c = k // 2
        p = k % 2
        numer = sc_ref[c, 0:N, p * 64:(p + 1) * 64]
        scol = 128 + 2 * c + p
        s = sc_ref[c, 0:N, scol:scol + 1]
        parts.append(numer / s)
    aggr = jnp.concatenate(parts, axis=-1)
    mx = jnp.max(aggr, axis=0, keepdims=True)
    ep = jnp.exp(aggr - mx)
    soft = ep / jnp.sum(ep, axis=0, keepdims=True)
    o_ref[...] = jnp.dot(soft, wo_ref[...], precision=_HI,
                         preferred_element_type=_F32) + bo_ref[...]


def kernel(edge_index, x, Wh, bh, a, ba, Wo, bo):
    loops = jnp.arange(N, dtype=edge_index.dtype)
    src = jnp.concatenate([edge_index[0], loops])
    dst = jnp.concatenate([edge_index[1], loops])
    src_p = jnp.concatenate([src, jnp.zeros((EPAD - EL,), jnp.int32)])
    dst_p = jnp.concatenate([dst, jnp.full((EPAD - EL,), N, jnp.int32)])
    src_c = src_p.reshape(-1, C)
    dst_c = dst_p.reshape(-1, C)
    idx2 = jnp.stack([src_c, dst_c], axis=1).reshape(-1, C)

    W1 = Wh.transpose(2, 0, 1).reshape(DX, K * DH)
    b1 = bh.reshape(1, K * DH)
    am = jnp.zeros((K * DH, 8), _F32)
    for k in range(K):
        am = am.at[k * DH:(k + 1) * DH, k].set(a[k, :DH])
        am = am.at[k * DH:(k + 1) * DH, 4 + k].set(a[k, DH:])

    h2, aux, t8 = pl.pallas_call(
        _tc1,
        out_shape=[
            jax.ShapeDtypeStruct((2 * N, 128), _F32),
            jax.ShapeDtypeStruct((NP, 8), _F32),
            jax.ShapeDtypeStruct((1, 8), _F32),
        ],
    )(x, W1, b1, am)

    adst16 = jnp.tile(aux[:, 0:4], (1, 4))
    asrc16 = jnp.tile(aux[:, 4:8], (1, 4))
    tk = t8[0, :4] + t8[0, 4:8] + ba
    tk = jnp.where(tk > 0, tk, 0.01 * tk)
    tpat = jnp.tile(tk, 4)
    bapat = jnp.tile(ba, 4)

    sc_out = _sc_edges(idx2, adst16, asrc16, h2, bapat, tpat)

    return pl.pallas_call(
        _tc2,
        out_shape=jax.ShapeDtypeStruct((N, DH), _F32),
    )(sc_out, Wo.T, bo[None, :])


# speedup vs baseline: 21.8312x; 1.1752x over previous
import functools

import jax
import jax.numpy as jnp
from jax import lax
from jax.experimental import pallas as pl
from jax.experimental.pallas import tpu as pltpu
from jax.experimental.pallas import tpu_sc as plsc

N = 10000
DX = 128
DH = 64
K = 4
E = 320000

NTILE = 16
NCORE = 2
C = 64
EL = E + N
PT = 20736
CHUNKS = PT // C
EPAD = PT * NTILE
RW = 144
NP = 10112
ZR = NP // NTILE

_F32 = jnp.float32
_HI = jax.lax.Precision.HIGHEST


def _tc1(x_ref, w_ref, b_ref, am_ref, h2_ref, aux_ref, t_ref):
    h = jnp.dot(x_ref[...], w_ref[...], precision=_HI,
                preferred_element_type=_F32) + b_ref[...]
    h2_ref[0:N, :] = h[:, 0:128]
    h2_ref[N:2 * N, :] = h[:, 128:256]
    aux = jnp.dot(h, am_ref[...], precision=_HI, preferred_element_type=_F32)
    aux_ref[0:N, :] = aux
    aux_ref[N:NP, :] = jnp.zeros((NP - N, 8), _F32)
    t_ref[...] = jnp.max(aux, axis=0, keepdims=True)


def _sc_body(idx_ref, ad_ref, as_ref, h2_ref, bap_ref, tp_ref,
             out_ref, idxA, idxB, offA, offB, sidxA, sidxB,
             auxdA, auxsA, auxdB, auxsB,
             hrowA, hrowB, stageA, stageB, exb, bap, tp, acc,
             gsA, gsB, ssA, ssB):
    c = lax.axis_index("c")
    s = lax.axis_index("s")

    def zrow_body(j, carry):
        for r in range(RW // 16):
            stageA[j, pl.ds(r * 16, 16)] = jnp.zeros((16,), _F32)
        return carry

    lax.fori_loop(0, C, zrow_body, 0)
    for i in range(ZR // C):
        pltpu.sync_copy(stageA, acc.at[pl.ds(s * ZR + i * C, C)])
    rem = ZR % C
    if rem:
        pltpu.sync_copy(stageA.at[pl.ds(0, rem)],
                        acc.at[pl.ds(s * ZR + (ZR // C) * C, rem)])
    pltpu.sync_copy(bap_ref, bap)
    pltpu.sync_copy(tp_ref, tp)
    plsc.subcore_barrier()

    ii = lax.iota(jnp.int32, 16)

    def run(hoff, hbase):
        srow, drow = 0, 1

        def load_idx(g, buf, off):
            pltpu.sync_copy(idx_ref.at[pl.ds((s * CHUNKS + g) * 2, 2)], buf)
            for r in range(C // 16):
                off[0, pl.ds(r * 16, 16)] = buf[srow, pl.ds(r * 16, 16)] + hbase

        def issue_gathers(buf, off, auxd, auxs, hrow, sem):
            pltpu.async_copy(h2_ref.at[off.at[0]], hrow, sem)
            pltpu.async_copy(ad_ref.at[buf.at[drow]], auxd, sem)
            pltpu.async_copy(as_ref.at[buf.at[srow]], auxs, sem)

        def wait_gathers(buf, off, auxd, auxs, hrow, sem):
            pltpu.make_async_copy(ad_ref.at[buf.at[drow]], auxd, sem).wait()
            pltpu.make_async_copy(as_ref.at[buf.at[srow]], auxs, sem).wait()
            pltpu.make_async_copy(h2_ref.at[off.at[0]], hrow, sem).wait()

        def compute(p, buf, sidx, auxd, auxs, hrow, stage, ssem):
            pl.when(p > 0)(
                lambda: pltpu.make_async_copy(
                    stage, acc.at[sidx.at[0]], ssem).wait())
            for r in range(C // 16):
                sidx[0, pl.ds(r * 16, 16)] = buf[drow, pl.ds(r * 16, 16)]

            ca = jnp.full((16,), hoff, jnp.int32)
            cb = jnp.full((16,), hoff + 1, jnp.int32)
            bavA = bap[hoff, :]
            bavB = bap[hoff + 1, :]
            tvA = tp[hoff, :]
            tvB = tp[hoff + 1, :]

            def group_body(i, carry2):
                rows = ii + i * 16
                zA = (plsc.load_gather(auxd, [rows, ca])
                      + plsc.load_gather(auxs, [rows, ca]) + bavA)
                zB = (plsc.load_gather(auxd, [rows, cb])
                      + plsc.load_gather(auxs, [rows, cb]) + bavB)
                zA = jnp.where(zA > 0, zA, 0.01 * zA) - tvA
                zB = jnp.where(zB > 0, zB, 0.01 * zB) - tvB
                exb[0:16] = jnp.exp(zA)
                exb[16:32] = jnp.exp(zB)
                for j2 in range(16):
                    j = i * 16 + j2
                    eav = plsc.load_gather(exb, [jnp.full((16,), j2,
                                                          jnp.int32)])
                    ebv = plsc.load_gather(exb, [jnp.full((16,), 16 + j2,
                                                          jnp.int32)])
                    for r in range(4):
                        stage[j, pl.ds(r * 16, 16)] = (
                            hrow[j, pl.ds(r * 16, 16)] * eav)
                    for r in range(4, 8):
                        stage[j, pl.ds(r * 16, 16)] = (
                            hrow[j, pl.ds(r * 16, 16)] * ebv)
                    stage[j, pl.ds(128, 16)] = jnp.where(
                        ii == hoff, eav, jnp.where(ii == hoff + 1, ebv, 0.0))
                return carry2

            lax.fori_loop(0, C // 16, group_body, 0)
            pltpu.async_copy(stage, acc.at[sidx.at[0]], ssem, add=True)

        load_idx(0, idxA, offA)
        issue_gathers(idxA, offA, auxdA, auxsA, hrowA, gsA)

        def pair_body(p, carry):
            wait_gathers(idxA, offA, auxdA, auxsA, hrowA, gsA)
            load_idx(2 * p + 1, idxB, offB)
            issue_gathers(idxB, offB, auxdB, auxsB, hrowB, gsB)
            compute(p, idxA, sidxA, auxdA, auxsA, hrowA, stageA, ssA)
            wait_gathers(idxB, offB, auxdB, auxsB, hrowB, gsB)

            def prefetch_a():
                load_idx(2 * p + 2, idxA, offA)
                issue_gathers(idxA, offA, auxdA, auxsA, hrowA, gsA)

            pl.when(p < CHUNKS // 2 - 1)(prefetch_a)
            compute(p, idxB, sidxB, auxdB, auxsB, hrowB, stageB, ssB)
            return carry

        lax.fori_loop(0, CHUNKS // 2, pair_body, 0)
        pltpu.make_async_copy(stageA, acc.at[sidxA.at[0]], ssA).wait()
        pltpu.make_async_copy(stageB, acc.at[sidxB.at[0]], ssB).wait()

    pl.when(c == 0)(lambda: run(0, 0))
    pl.when(c == 1)(lambda: run(2, N))
    plsc.subcore_barrier()
    pltpu.sync_copy(acc.at[pl.ds(s * ZR, ZR)],
                    out_ref.at[c, pl.ds(s * ZR, ZR)])


_sc_edges = functools.partial(
    pl.kernel,
    out_type=jax.ShapeDtypeStruct((NCORE, NP, RW), _F32),
    mesh=plsc.VectorSubcoreMesh(core_axis_name="c", subcore_axis_name="s"),
    scratch_types=[
        pltpu.VMEM((2, C), jnp.int32),
        pltpu.VMEM((2, C), jnp.int32),
        pltpu.VMEM((1, C), jnp.int32),
        pltpu.VMEM((1, C), jnp.int32),
        pltpu.VMEM((1, C), jnp.int32),
        pltpu.VMEM((1, C), jnp.int32),
        pltpu.VMEM((C, 16), _F32),
        pltpu.VMEM((C, 16), _F32),
        pltpu.VMEM((C, 16), _F32),
        pltpu.VMEM((C, 16), _F32),
        pltpu.VMEM((C, 128), _F32),
        pltpu.VMEM((C, 128), _F32),
        pltpu.VMEM((C, RW), _F32),
        pltpu.VMEM((C, RW), _F32),
        pltpu.VMEM((32,), _F32),
        pltpu.VMEM((4, 16), _F32),
        pltpu.VMEM((4, 16), _F32),
        pltpu.VMEM_SHARED((NP, RW), _F32),
        pltpu.SemaphoreType.DMA,
        pltpu.SemaphoreType.DMA,
        pltpu.SemaphoreType.DMA,
        pltpu.SemaphoreType.DMA,
    ],
    compiler_params=pltpu.CompilerParams(use_tc_tiling_on_sc=False,
                                         needs_layout_passes=False),
)(_sc_body)


def _tc2(sc_ref, wo_ref, bo_ref, o_ref):
    parts = []
    for k in range(K):
        c = k // 2
        p = k % 2
        numer = sc_ref[c, 0:N, p * 64:(p + 1) * 64]
        scol = 128 + 2 * c + p
        s = sc_ref[c, 0:N, scol:scol + 1]
        parts.append(numer / s)
    aggr = jnp.concatenate(parts, axis=-1)
    mx = jnp.max(aggr, axis=0, keepdims=True)
    ep = jnp.exp(aggr - mx)
    soft = ep / jnp.sum(ep, axis=0, keepdims=True)
    o_ref[...] = jnp.dot(soft, wo_ref[...], precision=_HI,
                         preferred_element_type=_F32) + bo_ref[...]


def kernel(edge_index, x, Wh, bh, a, ba, Wo, bo):
    loops = jnp.arange(N, dtype=edge_index.dtype)
    src = jnp.concatenate([edge_index[0], loops])
    dst = jnp.concatenate([edge_index[1], loops])
    src_p = jnp.concatenate([src, jnp.zeros((EPAD - EL,), jnp.int32)])
    dst_p = jnp.concatenate([dst, jnp.full((EPAD - EL,), N, jnp.int32)])
    src_c = src_p.reshape(-1, C)
    dst_c = dst_p.reshape(-1, C)
    idx2 = jnp.stack([src_c, dst_c], axis=1).reshape(-1, C)

    W1 = Wh.transpose(2, 0, 1).reshape(DX, K * DH)
    b1 = bh.reshape(1, K * DH)
    am = jnp.zeros((K * DH, 8), _F32)
    for k in range(K):
        am = am.at[k * DH:(k + 1) * DH, k].set(a[k, :DH])
        am = am.at[k * DH:(k + 1) * DH, 4 + k].set(a[k, DH:])

    h2, aux, t8 = pl.pallas_call(
        _tc1,
        out_shape=[
            jax.ShapeDtypeStruct((2 * N, 128), _F32),
            jax.ShapeDtypeStruct((NP, 8), _F32),
            jax.ShapeDtypeStruct((1, 8), _F32),
        ],
    )(x, W1, b1, am)

    adst16 = jnp.tile(aux[:, 0:4], (1, 4))
    asrc16 = jnp.tile(aux[:, 4:8], (1, 4))
    tk = t8[0, :4] + t8[0, 4:8] + ba
    tk = jnp.where(tk > 0, tk, 0.01 * tk)
    tpat = jnp.tile(tk[:, None], (1, 16))
    bapat = jnp.tile(ba[:, None], (1, 16))

    sc_out = _sc_edges(idx2, adst16, asrc16, h2, bapat, tpat)

    return pl.pallas_call(
        _tc2,
        out_shape=jax.ShapeDtypeStruct((N, DH), _F32),
    )(sc_out, Wo.T, bo[None, :])


# speedup vs baseline: 28.6682x; 1.3132x over previous
import functools

import jax
import jax.numpy as jnp
from jax import lax
from jax.experimental import pallas as pl
from jax.experimental.pallas import tpu as pltpu
from jax.experimental.pallas import tpu_sc as plsc

N = 10000
DX = 128
DH = 64
K = 4
E = 320000

NTILE = 16
NCORE = 2
C = 64
EL = E + N
PT = 20736
CHUNKS = PT // C
EPAD = PT * NTILE
RW = 144
NP = 10112
ZR = NP // NTILE

_F32 = jnp.float32
_HI = jax.lax.Precision.HIGHEST


def _tc1(x_ref, w_ref, b_ref, am_ref, h2_ref, aux_ref, t_ref):
    h = jnp.dot(x_ref[...], w_ref[...], precision=_HI,
                preferred_element_type=_F32) + b_ref[...]
    h2_ref[0:N, :] = h[:, 0:128]
    h2_ref[N:2 * N, :] = h[:, 128:256]
    aux = jnp.dot(h, am_ref[...], precision=_HI, preferred_element_type=_F32)
    aux_ref[0:N, :] = aux
    aux_ref[N:NP, :] = jnp.zeros((NP - N, 8), _F32)
    t_ref[...] = jnp.max(aux, axis=0, keepdims=True)


def _sc_body(idx_ref, aux_ref, h2_ref, bap_ref, tp_ref,
             out_ref, idxA, idxB, offA, offB, sidxA, sidxB,
             auxdA, auxsA, auxdB, auxsB,
             hrowA, hrowB, stageA, stageB, exb, bap, tp, acc,
             gsA, gsB, ssA, ssB):
    c = lax.axis_index("c")
    s = lax.axis_index("s")

    def zrow_body(j, carry):
        for r in range(RW // 16):
            stageA[j, pl.ds(r * 16, 16)] = jnp.zeros((16,), _F32)
        return carry

    lax.fori_loop(0, C, zrow_body, 0)
    for i in range(ZR // C):
        pltpu.sync_copy(stageA, acc.at[pl.ds(s * ZR + i * C, C)])
    rem = ZR % C
    if rem:
        pltpu.sync_copy(stageA.at[pl.ds(0, rem)],
                        acc.at[pl.ds(s * ZR + (ZR // C) * C, rem)])
    pltpu.sync_copy(bap_ref, bap)
    pltpu.sync_copy(tp_ref, tp)
    plsc.subcore_barrier()

    ii = lax.iota(jnp.int32, 16)

    def run(hoff, hbase):
        srow, drow = 0, 1

        def load_idx(g, buf, off):
            pltpu.sync_copy(idx_ref.at[pl.ds((s * CHUNKS + g) * 2, 2)], buf)
            for r in range(C // 16):
                off[0, pl.ds(r * 16, 16)] = buf[srow, pl.ds(r * 16, 16)] + hbase

        def issue_gathers(buf, off, auxd, auxs, hrow, sem):
            pltpu.async_copy(h2_ref.at[off.at[0]], hrow, sem)
            pltpu.async_copy(aux_ref.at[buf.at[drow]], auxd, sem)
            pltpu.async_copy(aux_ref.at[buf.at[srow]], auxs, sem)

        def wait_gathers(buf, off, auxd, auxs, hrow, sem):
            pltpu.make_async_copy(aux_ref.at[buf.at[drow]], auxd, sem).wait()
            pltpu.make_async_copy(aux_ref.at[buf.at[srow]], auxs, sem).wait()
            pltpu.make_async_copy(h2_ref.at[off.at[0]], hrow, sem).wait()

        def compute(p, buf, sidx, auxd, auxs, hrow, stage, ssem):
            pl.when(p > 0)(
                lambda: pltpu.make_async_copy(
                    stage, acc.at[sidx.at[0]], ssem).wait())
            for r in range(C // 16):
                sidx[0, pl.ds(r * 16, 16)] = buf[drow, pl.ds(r * 16, 16)]

            ca = jnp.full((16,), hoff, jnp.int32)
            cb = jnp.full((16,), hoff + 1, jnp.int32)
            sa = jnp.full((16,), hoff + 4, jnp.int32)
            sb = jnp.full((16,), hoff + 5, jnp.int32)
            bavA = bap[hoff, :]
            bavB = bap[hoff + 1, :]
            tvA = tp[hoff, :]
            tvB = tp[hoff + 1, :]

            def group_body(i, carry2):
                rows = ii + i * 16
                zA = (plsc.load_gather(auxd, [rows, ca])
                      + plsc.load_gather(auxs, [rows, sa]) + bavA)
                zB = (plsc.load_gather(auxd, [rows, cb])
                      + plsc.load_gather(auxs, [rows, sb]) + bavB)
                zA = jnp.where(zA > 0, zA, 0.01 * zA) - tvA
                zB = jnp.where(zB > 0, zB, 0.01 * zB) - tvB
                exb[0:16] = jnp.exp(zA)
                exb[16:32] = jnp.exp(zB)
                for j2 in range(16):
                    j = i * 16 + j2
                    eav = plsc.load_gather(exb, [jnp.full((16,), j2,
                                                          jnp.int32)])
                    ebv = plsc.load_gather(exb, [jnp.full((16,), 16 + j2,
                                                          jnp.int32)])
                    for r in range(4):
                        ev = eav if r < 2 else ebv
                        ha, hb = plsc.unpack(
                            hrow[j, pl.ds(r * 32, 32)],
                            format=plsc.PackFormat.INTERLEAVED)
                        stage[j, pl.ds(r * 32, 16)] = ha * ev
                        stage[j, pl.ds(r * 32 + 16, 16)] = hb * ev
                    stage[j, pl.ds(128, 16)] = jnp.where(
                        ii == hoff, eav, jnp.where(ii == hoff + 1, ebv, 0.0))
                return carry2

            lax.fori_loop(0, C // 16, group_body, 0)
            pltpu.async_copy(stage, acc.at[sidx.at[0]], ssem, add=True)

        load_idx(0, idxA, offA)
        issue_gathers(idxA, offA, auxdA, auxsA, hrowA, gsA)

        def pair_body(p, carry):
            wait_gathers(idxA, offA, auxdA, auxsA, hrowA, gsA)
            load_idx(2 * p + 1, idxB, offB)
            issue_gathers(idxB, offB, auxdB, auxsB, hrowB, gsB)
            compute(p, idxA, sidxA, auxdA, auxsA, hrowA, stageA, ssA)
            wait_gathers(idxB, offB, auxdB, auxsB, hrowB, gsB)

            def prefetch_a():
                load_idx(2 * p + 2, idxA, offA)
                issue_gathers(idxA, offA, auxdA, auxsA, hrowA, gsA)

            pl.when(p < CHUNKS // 2 - 1)(prefetch_a)
            compute(p, idxB, sidxB, auxdB, auxsB, hrowB, stageB, ssB)
            return carry

        lax.fori_loop(0, CHUNKS // 2, pair_body, 0)
        pltpu.make_async_copy(stageA, acc.at[sidxA.at[0]], ssA).wait()
        pltpu.make_async_copy(stageB, acc.at[sidxB.at[0]], ssB).wait()

    pl.when(c == 0)(lambda: run(0, 0))
    pl.when(c == 1)(lambda: run(2, N))
    plsc.subcore_barrier()
    pltpu.sync_copy(acc.at[pl.ds(s * ZR, ZR)],
                    out_ref.at[c, pl.ds(s * ZR, ZR)])


_sc_edges = functools.partial(
    pl.kernel,
    out_type=jax.ShapeDtypeStruct((NCORE, NP, RW), _F32),
    mesh=plsc.VectorSubcoreMesh(core_axis_name="c", subcore_axis_name="s"),
    scratch_types=[
        pltpu.VMEM((2, C), jnp.int32),
        pltpu.VMEM((2, C), jnp.int32),
        pltpu.VMEM((1, C), jnp.int32),
        pltpu.VMEM((1, C), jnp.int32),
        pltpu.VMEM((1, C), jnp.int32),
        pltpu.VMEM((1, C), jnp.int32),
        pltpu.VMEM((C, 8), _F32),
        pltpu.VMEM((C, 8), _F32),
        pltpu.VMEM((C, 8), _F32),
        pltpu.VMEM((C, 8), _F32),
        pltpu.VMEM((C, 128), jnp.bfloat16),
        pltpu.VMEM((C, 128), jnp.bfloat16),
        pltpu.VMEM((C, RW), _F32),
        pltpu.VMEM((C, RW), _F32),
        pltpu.VMEM((32,), _F32),
        pltpu.VMEM((4, 16), _F32),
        pltpu.VMEM((4, 16), _F32),
        pltpu.VMEM_SHARED((NP, RW), _F32),
        pltpu.SemaphoreType.DMA,
        pltpu.SemaphoreType.DMA,
        pltpu.SemaphoreType.DMA,
        pltpu.SemaphoreType.DMA,
    ],
    compiler_params=pltpu.CompilerParams(use_tc_tiling_on_sc=False,
                                         needs_layout_passes=False),
)(_sc_body)


def _tc2(sc_ref, wo_ref, bo_ref, o_ref):
    parts = []
    for k in range(K):
        c = k // 2
        p = k % 2
        numer = sc_ref[c, 0:N, p * 64:(p + 1) * 64]
        scol = 128 + 2 * c + p
        s = sc_ref[c, 0:N, scol:scol + 1]
        parts.append(numer / s)
    aggr = jnp.concatenate(parts, axis=-1)
    mx = jnp.max(aggr, axis=0, keepdims=True)
    ep = jnp.exp(aggr - mx)
    soft = ep / jnp.sum(ep, axis=0, keepdims=True)
    o_ref[...] = jnp.dot(soft, wo_ref[...], precision=_HI,
                         preferred_element_type=_F32) + bo_ref[...]


def kernel(edge_index, x, Wh, bh, a, ba, Wo, bo):
    loops = jnp.arange(N, dtype=edge_index.dtype)
    src = jnp.concatenate([edge_index[0], loops])
    dst = jnp.concatenate([edge_index[1], loops])
    src_p = jnp.concatenate([src, jnp.zeros((EPAD - EL,), jnp.int32)])
    dst_p = jnp.concatenate([dst, jnp.full((EPAD - EL,), N, jnp.int32)])
    src_c = src_p.reshape(-1, C)
    dst_c = dst_p.reshape(-1, C)
    idx2 = jnp.stack([src_c, dst_c], axis=1).reshape(-1, C)

    W1 = Wh.transpose(2, 0, 1).reshape(DX, K * DH)
    b1 = bh.reshape(1, K * DH)
    am = jnp.zeros((K * DH, 8), _F32)
    for k in range(K):
        am = am.at[k * DH:(k + 1) * DH, k].set(a[k, :DH])
        am = am.at[k * DH:(k + 1) * DH, 4 + k].set(a[k, DH:])

    h2, aux, t8 = pl.pallas_call(
        _tc1,
        out_shape=[
            jax.ShapeDtypeStruct((2 * N, 128), _F32),
            jax.ShapeDtypeStruct((NP, 8), _F32),
            jax.ShapeDtypeStruct((1, 8), _F32),
        ],
    )(x, W1, b1, am)

    tk = t8[0, :4] + t8[0, 4:8] + ba
    tk = jnp.where(tk > 0, tk, 0.01 * tk)
    tpat = jnp.tile(tk[:, None], (1, 16))
    bapat = jnp.tile(ba[:, None], (1, 16))

    base = jnp.arange(4) * 32
    lane = jnp.arange(16)
    inner = jnp.stack([lane, lane + 16], axis=1).reshape(-1)
    perm = (base[:, None] + inner[None, :]).reshape(-1)
    h2bf = h2[:, perm].astype(jnp.bfloat16)

    sc_out = _sc_edges(idx2, aux, h2bf, bapat, tpat)

    return pl.pallas_call(
        _tc2,
        out_shape=jax.ShapeDtypeStruct((N, DH), _F32),
    )(sc_out, Wo.T, bo[None, :])


# speedup vs baseline: 29.6999x; 1.0360x over previous
import functools

import numpy as np

import jax
import jax.numpy as jnp
from jax import lax
from jax.experimental import pallas as pl
from jax.experimental.pallas import tpu as pltpu
from jax.experimental.pallas import tpu_sc as plsc

N = 10000
DX = 128
DH = 64
K = 4
E = 320000

NTILE = 16
NCORE = 2
C = 64
EL = E + N
PT = 20736
CHUNKS = PT // C
EPAD = PT * NTILE
RW = 144
NP = 10112
ZR = NP // NTILE

_F32 = jnp.float32
_HI = jax.lax.Precision.HIGHEST


def _tc1(x_ref, w_ref, b_ref, am_ref, h2_ref, aux_ref, t_ref):
    h = jnp.dot(x_ref[...], w_ref[...], precision=_HI,
                preferred_element_type=_F32) + b_ref[...]
    h2_ref[0:N, :] = h[:, 0:128].astype(jnp.bfloat16)
    h2_ref[N:2 * N, :] = h[:, 128:256].astype(jnp.bfloat16)
    aux = jnp.dot(h, am_ref[...], precision=_HI, preferred_element_type=_F32)
    aux_ref[0:N, :] = aux
    aux_ref[N:NP, :] = jnp.zeros((NP - N, 8), _F32)
    t_ref[...] = jnp.max(aux, axis=0, keepdims=True)


def _sc_body(idx_ref, aux_ref, h2_ref, bap_ref, tp_ref,
             out_ref, idxA, idxB, offA, offB, sidxA, sidxB,
             auxdA, auxsA, auxdB, auxsB,
             hrowA, hrowB, stageA, stageB, exb, bap, tp, acc,
             gsA, gsB, ssA, ssB):
    c = lax.axis_index("c")
    s = lax.axis_index("s")

    def zrow_body(j, carry):
        for r in range(RW // 16):
            stageA[j, pl.ds(r * 16, 16)] = jnp.zeros((16,), _F32)
        return carry

    lax.fori_loop(0, C, zrow_body, 0)
    for i in range(ZR // C):
        pltpu.sync_copy(stageA, acc.at[pl.ds(s * ZR + i * C, C)])
    rem = ZR % C
    if rem:
        pltpu.sync_copy(stageA.at[pl.ds(0, rem)],
                        acc.at[pl.ds(s * ZR + (ZR // C) * C, rem)])
    pltpu.sync_copy(bap_ref, bap)
    pltpu.sync_copy(tp_ref, tp)
    plsc.subcore_barrier()

    ii = lax.iota(jnp.int32, 16)

    def run(hoff, hbase):
        srow, drow = 0, 1

        def load_idx(g, buf, off):
            pltpu.sync_copy(idx_ref.at[pl.ds((s * CHUNKS + g) * 2, 2)], buf)
            for r in range(C // 16):
                off[0, pl.ds(r * 16, 16)] = buf[srow, pl.ds(r * 16, 16)] + hbase

        def issue_gathers(buf, off, auxd, auxs, hrow, sem):
            pltpu.async_copy(h2_ref.at[off.at[0]], hrow, sem)
            pltpu.async_copy(aux_ref.at[buf.at[drow]], auxd, sem)
            pltpu.async_copy(aux_ref.at[buf.at[srow]], auxs, sem)

        def wait_gathers(buf, off, auxd, auxs, hrow, sem):
            pltpu.make_async_copy(aux_ref.at[buf.at[drow]], auxd, sem).wait()
            pltpu.make_async_copy(aux_ref.at[buf.at[srow]], auxs, sem).wait()
            pltpu.make_async_copy(h2_ref.at[off.at[0]], hrow, sem).wait()

        def compute(p, buf, sidx, auxd, auxs, hrow, stage, ssem):
            pl.when(p > 0)(
                lambda: pltpu.make_async_copy(
                    stage, acc.at[sidx.at[0]], ssem).wait())
            for r in range(C // 16):
                sidx[0, pl.ds(r * 16, 16)] = buf[drow, pl.ds(r * 16, 16)]

            ca = jnp.full((16,), hoff, jnp.int32)
            cb = jnp.full((16,), hoff + 1, jnp.int32)
            sa = jnp.full((16,), hoff + 4, jnp.int32)
            sb = jnp.full((16,), hoff + 5, jnp.int32)
            bavA = bap[hoff, :]
            bavB = bap[hoff + 1, :]
            tvA = tp[hoff, :]
            tvB = tp[hoff + 1, :]

            def group_body(i, carry2):
                rows = ii + i * 16
                zA = (plsc.load_gather(auxd, [rows, ca])
                      + plsc.load_gather(auxs, [rows, sa]) + bavA)
                zB = (plsc.load_gather(auxd, [rows, cb])
                      + plsc.load_gather(auxs, [rows, sb]) + bavB)
                zA = jnp.where(zA > 0, zA, 0.01 * zA) - tvA
                zB = jnp.where(zB > 0, zB, 0.01 * zB) - tvB
                exb[0:16] = jnp.exp(zA)
                exb[16:32] = jnp.exp(zB)
                for j2 in range(16):
                    j = i * 16 + j2
                    eav = plsc.load_gather(exb, [jnp.full((16,), j2,
                                                          jnp.int32)])
                    ebv = plsc.load_gather(exb, [jnp.full((16,), 16 + j2,
                                                          jnp.int32)])
                    for r in range(4):
                        ev = eav if r < 2 else ebv
                        ha, hb = plsc.unpack(
                            hrow[j, pl.ds(r * 32, 32)],
                            format=plsc.PackFormat.INTERLEAVED)
                        stage[j, pl.ds(r * 32, 16)] = ha * ev
                        stage[j, pl.ds(r * 32 + 16, 16)] = hb * ev
                    stage[j, pl.ds(128, 16)] = jnp.where(
                        ii == hoff, eav, jnp.where(ii == hoff + 1, ebv, 0.0))
                return carry2

            lax.fori_loop(0, C // 16, group_body, 0)
            pltpu.async_copy(stage, acc.at[sidx.at[0]], ssem, add=True)

        load_idx(0, idxA, offA)
        issue_gathers(idxA, offA, auxdA, auxsA, hrowA, gsA)

        def pair_body(p, carry):
            wait_gathers(idxA, offA, auxdA, auxsA, hrowA, gsA)
            load_idx(2 * p + 1, idxB, offB)
            issue_gathers(idxB, offB, auxdB, auxsB, hrowB, gsB)
            compute(p, idxA, sidxA, auxdA, auxsA, hrowA, stageA, ssA)
            wait_gathers(idxB, offB, auxdB, auxsB, hrowB, gsB)

            def prefetch_a():
                load_idx(2 * p + 2, idxA, offA)
                issue_gathers(idxA, offA, auxdA, auxsA, hrowA, gsA)

            pl.when(p < CHUNKS // 2 - 1)(prefetch_a)
            compute(p, idxB, sidxB, auxdB, auxsB, hrowB, stageB, ssB)
            return carry

        lax.fori_loop(0, CHUNKS // 2, pair_body, 0)
        pltpu.make_async_copy(stageA, acc.at[sidxA.at[0]], ssA).wait()
        pltpu.make_async_copy(stageB, acc.at[sidxB.at[0]], ssB).wait()

    pl.when(c == 0)(lambda: run(0, 0))
    pl.when(c == 1)(lambda: run(2, N))
    plsc.subcore_barrier()
    pltpu.sync_copy(acc.at[pl.ds(s * ZR, ZR)],
                    out_ref.at[c, pl.ds(s * ZR, ZR)])


_sc_edges = functools.partial(
    pl.kernel,
    out_type=jax.ShapeDtypeStruct((NCORE, NP, RW), _F32),
    mesh=plsc.VectorSubcoreMesh(core_axis_name="c", subcore_axis_name="s"),
    scratch_types=[
        pltpu.VMEM((2, C), jnp.int32),
        pltpu.VMEM((2, C), jnp.int32),
        pltpu.VMEM((1, C), jnp.int32),
        pltpu.VMEM((1, C), jnp.int32),
        pltpu.VMEM((1, C), jnp.int32),
        pltpu.VMEM((1, C), jnp.int32),
        pltpu.VMEM((C, 8), _F32),
        pltpu.VMEM((C, 8), _F32),
        pltpu.VMEM((C, 8), _F32),
        pltpu.VMEM((C, 8), _F32),
        pltpu.VMEM((C, 128), jnp.bfloat16),
        pltpu.VMEM((C, 128), jnp.bfloat16),
        pltpu.VMEM((C, RW), _F32),
        pltpu.VMEM((C, RW), _F32),
        pltpu.VMEM((32,), _F32),
        pltpu.VMEM((4, 16), _F32),
        pltpu.VMEM((4, 16), _F32),
        pltpu.VMEM_SHARED((NP, RW), _F32),
        pltpu.SemaphoreType.DMA,
        pltpu.SemaphoreType.DMA,
        pltpu.SemaphoreType.DMA,
        pltpu.SemaphoreType.DMA,
    ],
    compiler_params=pltpu.CompilerParams(use_tc_tiling_on_sc=False,
                                         needs_layout_passes=False),
)(_sc_body)


def _tc2(sc_ref, wo_ref, bo_ref, o_ref):
    parts = []
    for k in range(K):
        c = k // 2
        p = k % 2
        numer = sc_ref[c, 0:N, p * 64:(p + 1) * 64]
        scol = 128 + 2 * c + p
        s = sc_ref[c, 0:N, scol:scol + 1]
        parts.append(numer / s)
    aggr = jnp.concatenate(parts, axis=-1)
    mx = jnp.max(aggr, axis=0, keepdims=True)
    ep = jnp.exp(aggr - mx)
    soft = ep / jnp.sum(ep, axis=0, keepdims=True)
    o_ref[...] = jnp.dot(soft, wo_ref[...], precision=_HI,
                         preferred_element_type=_F32) + bo_ref[...]


def kernel(edge_index, x, Wh, bh, a, ba, Wo, bo):
    loops = jnp.arange(N, dtype=edge_index.dtype)
    src = jnp.concatenate([edge_index[0], loops])
    dst = jnp.concatenate([edge_index[1], loops])
    src_p = jnp.concatenate([src, jnp.zeros((EPAD - EL,), jnp.int32)])
    dst_p = jnp.concatenate([dst, jnp.full((EPAD - EL,), N, jnp.int32)])
    src_c = src_p.reshape(-1, C)
    dst_c = dst_p.reshape(-1, C)
    idx2 = jnp.stack([src_c, dst_c], axis=1).reshape(-1, C)

    W1 = Wh.transpose(2, 0, 1).reshape(DX, K * DH)
    b1 = bh.reshape(1, K * DH)
    am = jnp.zeros((K * DH, 8), _F32)
    for k in range(K):
        am = am.at[k * DH:(k + 1) * DH, k].set(a[k, :DH])
        am = am.at[k * DH:(k + 1) * DH, 4 + k].set(a[k, DH:])

    h2bf, aux, t8 = pl.pallas_call(
        _tc1,
        out_shape=[
            jax.ShapeDtypeStruct((2 * N, 128), jnp.bfloat16),
            jax.ShapeDtypeStruct((NP, 8), _F32),
            jax.ShapeDtypeStruct((1, 8), _F32),
        ],
    )(x, W1, b1, am)

    tk = t8[0, :4] + t8[0, 4:8] + ba
    tk = jnp.where(tk > 0, tk, 0.01 * tk)
    tpat = jnp.tile(tk[:, None], (1, 16))
    bapat = jnp.tile(ba[:, None], (1, 16))

    sc_out = _sc_edges(idx2, aux, h2bf, bapat, tpat)

    ev = 2 * np.arange(16)
    blk = np.concatenate([ev, ev + 1])
    perm_rows = np.concatenate([32 * r + blk for r in range(K * DH // 32)])
    wo2p = Wo.T[perm_rows]

    return pl.pallas_call(
        _tc2,
        out_shape=jax.ShapeDtypeStruct((N, DH), _F32),
    )(sc_out, wo2p, bo[None, :])
